# baseline (speedup 1.0000x reference)
import functools

import jax
import jax.numpy as jnp
from jax import lax
from jax.experimental import pallas as pl
from jax.experimental.pallas import tpu as pltpu
from jax.experimental.pallas import tpu_sc as plsc

B, S, D = 1, 2048, 768
E, K, F = 16, 2, 384
T = B * S
P = K * T

BT = 128
NT = P // BT + E
NSLOT = NT * BT

NC, NS = 2, 16
NW = NC * NS
TPW = T // NW
CH = 512


def _pack_bf16(v):
    bits = lax.bitcast_convert_type(v, jnp.int32)
    rnd = bits + 0x7FFF + jnp.bitwise_and(
        lax.shift_right_logical(bits, 16), 1)
    bf = lax.shift_right_logical(rnd, 16)
    lo = bf[:, :D // 2]
    hi = bf[:, D // 2:]
    return jnp.bitwise_or(lo, lax.shift_left(hi, 16))


def _route_body(x_ref, rw_ref, pos_ref, w_ref, texp_ref, tvalid_ref,
                tslot_ref, tgrp_ref, tfirst_ref, gexp_ref, gvalid_ref,
                xpk_ref):
    x = x_ref[...]
    xpk_ref[...] = _pack_bf16(x)
    ids = lax.broadcasted_iota(jnp.int32, (T, E), 1)
    logits = lax.dot_general(x, rw_ref[...], (((1,), (1,)), ((), ())),
                             preferred_element_type=jnp.float32)
    m = jnp.max(logits, axis=-1, keepdims=True)
    p = jnp.exp(logits - m)
    p = p / jnp.sum(p, axis=-1, keepdims=True)
    m1 = jnp.max(p, axis=-1, keepdims=True)
    i1 = jnp.min(jnp.where(p == m1, ids, E), axis=-1, keepdims=True)
    p2 = jnp.where(ids == i1, -1.0, p)
    m2 = jnp.max(p2, axis=-1, keepdims=True)
    i2 = jnp.min(jnp.where(p2 == m2, ids, E), axis=-1, keepdims=True)
    s = m1 + m2 + 1e-20
    w_ref[...] = jnp.concatenate([m1 / s, m2 / s], axis=1)

    cnt = (jnp.where(ids == i1, 1.0, 0.0)
           + jnp.where(ids == i2, 1.0, 0.0))
    r_io = lax.broadcasted_iota(jnp.int32, (CH, CH), 0)
    c_io = lax.broadcasted_iota(jnp.int32, (CH, CH), 1)
    tri = jnp.where(r_io >= c_io, 1.0, 0.0)
    carry = jnp.zeros((1, E), jnp.float32)
    cex_chunks = []
    for ci in range(T // CH):
        blk = cnt[ci * CH:(ci + 1) * CH]
        inc = lax.dot_general(tri, blk, (((1,), (0,)), ((), ())),
                              preferred_element_type=jnp.float32) + carry
        cex_chunks.append(inc - blk)
        carry = inc[CH - 1:CH, :]
    cex = jnp.concatenate(cex_chunks, axis=0)
    counts = carry

    tiles = (counts.astype(jnp.int32) + (BT - 1)) // BT
    tiles_f = tiles.astype(jnp.float32)
    er_io = lax.broadcasted_iota(jnp.int32, (E, E), 0)
    ec_io = lax.broadcasted_iota(jnp.int32, (E, E), 1)
    tril = jnp.where(er_io < ec_io, 1.0, 0.0)
    tilebase = lax.dot_general(tiles_f, tril, (((1,), (0,)), ((), ())),
                               preferred_element_type=jnp.float32)
    slotbase = tilebase * BT

    def _sel(mat, idx_t1):
        src = mat if mat.shape[0] == T else jnp.broadcast_to(mat, (T, E))
        return jnp.sum(jnp.where(ids == idx_t1, src, 0.0),
                       axis=-1, keepdims=True)

    pos0 = _sel(slotbase, i1) + _sel(cex, i1)
    pos1 = _sel(slotbase, i2) + _sel(cex, i2)
    pos_ref[...] = jnp.concatenate([pos0, pos1], axis=1).astype(jnp.int32)

    n_io = lax.broadcasted_iota(jnp.int32, (NT, E), 0).astype(jnp.float32)
    e_io = lax.broadcasted_iota(jnp.int32, (NT, E), 1).astype(jnp.float32)
    tb = jnp.broadcast_to(tilebase, (NT, E))
    tl = jnp.broadcast_to(tiles_f, (NT, E))
    inrange = jnp.logical_and(n_io >= tb, n_io < tb + tl)
    te = jnp.sum(jnp.where(inrange, e_io, 0.0), axis=-1, keepdims=True)
    total_tiles = jnp.sum(tiles_f)
    e1_io = lax.broadcasted_iota(jnp.int32, (1, E), 1).astype(jnp.float32)
    te_last = jnp.max(jnp.where(tiles_f > 0, e1_io, 0.0))
    valid = n_io[:, :1] < total_tiles
    texp_ref[...] = jnp.where(valid, te, te_last).astype(jnp.int32)
    tvalid_ref[...] = valid.astype(jnp.int32)
    tslot_ref[...] = jnp.where(valid, n_io[:, :1],
                               total_tiles - 1.0).astype(jnp.int32)

    used = jnp.where(counts > 0, 1.0, 0.0)
    gidx = lax.dot_general(used, tril, (((1,), (0,)), ((), ())),
                           preferred_element_type=jnp.float32)
    ngroups = jnp.sum(used)
    g_io = lax.broadcasted_iota(jnp.int32, (E, E), 0).astype(jnp.float32)
    e2_io = lax.broadcasted_iota(jnp.int32, (E, E), 1).astype(jnp.float32)
    cond = jnp.logical_and(jnp.broadcast_to(gidx, (E, E)) == g_io,
                           jnp.broadcast_to(used, (E, E)) > 0)
    gexp_ref[...] = jnp.sum(jnp.where(cond, e2_io, 0.0), axis=-1,
                            keepdims=True).astype(jnp.int32)
    gvalid_ref[...] = (g_io[:, :1] < ngroups).astype(jnp.int32)
    tgrp_ref[...] = jnp.sum(
        jnp.where(inrange, jnp.broadcast_to(gidx, (NT, E)), 0.0),
        axis=-1, keepdims=True).astype(jnp.int32)
    tfirst_ref[...] = jnp.sum(
        jnp.where(jnp.logical_and(inrange, n_io == tb), 1.0, 0.0),
        axis=-1, keepdims=True).astype(jnp.int32)


def _route(hs, router_w):
    return pl.pallas_call(
        _route_body,
        out_shape=[
            jax.ShapeDtypeStruct((T, K), jnp.int32),
            jax.ShapeDtypeStruct((T, K), jnp.float32),
            jax.ShapeDtypeStruct((NT, 1), jnp.int32),
            jax.ShapeDtypeStruct((NT, 1), jnp.int32),
            jax.ShapeDtypeStruct((NT, 1), jnp.int32),
            jax.ShapeDtypeStruct((NT, 1), jnp.int32),
            jax.ShapeDtypeStruct((NT, 1), jnp.int32),
            jax.ShapeDtypeStruct((E, 1), jnp.int32),
            jax.ShapeDtypeStruct((E, 1), jnp.int32),
            jax.ShapeDtypeStruct((T, D // 2), jnp.int32),
        ],
    )(hs, router_w)


def _dispatch(xpk, pos):
    mesh = plsc.VectorSubcoreMesh(core_axis_name="c", subcore_axis_name="s")

    @functools.partial(
        pl.kernel, mesh=mesh,
        out_type=jax.ShapeDtypeStruct((NSLOT, D // 2), jnp.int32),
        scratch_types=[
            pltpu.VMEM((TPW, D // 2), jnp.int32),
            pltpu.VMEM((TPW, K), jnp.int32),
            pltpu.VMEM((TPW,), jnp.int32),
            pltpu.VMEM((TPW,), jnp.int32),
            pltpu.SemaphoreType.DMA,
            pltpu.SemaphoreType.DMA,
        ],
        compiler_params=pltpu.CompilerParams(needs_layout_passes=False),
    )
    def disp(x_hbm, pos_hbm, xs_hbm, rows_v, praw_v, idx0_v, idx1_v, s0, s1):
        wid = lax.axis_index("s") * NC + lax.axis_index("c")
        tbase = wid * TPW
        pltpu.sync_copy(x_hbm.at[pl.ds(tbase, TPW)], rows_v)
        pltpu.sync_copy(pos_hbm.at[pl.ds(tbase, TPW)], praw_v)
        lane = lax.broadcasted_iota(jnp.int32, (16,), 0)
        c0 = jnp.zeros((16,), jnp.int32)
        c1 = jnp.ones((16,), jnp.int32)
        for c in range(TPW // 16):
            idx0_v[pl.ds(16 * c, 16)] = plsc.load_gather(
                praw_v, [16 * c + lane, c0])
            idx1_v[pl.ds(16 * c, 16)] = plsc.load_gather(
                praw_v, [16 * c + lane, c1])
        cp0 = pltpu.async_copy(rows_v, xs_hbm.at[idx0_v], s0)
        cp1 = pltpu.async_copy(rows_v, xs_hbm.at[idx1_v], s1)
        cp0.wait()
        cp1.wait()

    return disp(xpk, pos)


NBUF = 6


def _gmm_body(tvalid_ref, tslot_ref, tgrp_ref, tfirst_ref, gexp_ref,
              gvalid_ref, xs_ref, wg_hbm, wd_hbm, ys_ref, wgb, wdb, sg, sd):
    i = pl.program_id(0)

    def issue(g):
        gc = jnp.minimum(g, E - 1)
        e = gexp_ref[gc, 0]
        b = lax.rem(gc, NBUF)

        @pl.when(jnp.logical_and(g < E, gvalid_ref[gc, 0] == 1))
        def _():
            pltpu.make_async_copy(wg_hbm.at[e], wgb.at[b], sg.at[b]).start()
            pltpu.make_async_copy(wd_hbm.at[e], wdb.at[b], sd.at[b]).start()

    @pl.when(i == 0)
    def _():
        for g0 in range(NBUF - 1):
            issue(jnp.int32(g0))

    @pl.when(tvalid_ref[i, 0] > 0)
    def _():
        g = tgrp_ref[i, 0]
        b = lax.rem(g, NBUF)
        e = gexp_ref[g, 0]

        @pl.when(tfirst_ref[i, 0] == 1)
        def _():
            pltpu.make_async_copy(wg_hbm.at[e], wgb.at[b], sg.at[b]).wait()
            pltpu.make_async_copy(wd_hbm.at[e], wdb.at[b], sd.at[b]).wait()
            issue(g + NBUF - 1)

        wg = wgb[pl.ds(b, 1)][0]
        wd = wdb[pl.ds(b, 1)][0]
        w32 = xs_ref[...]
        xlo = lax.bitcast_convert_type(lax.shift_left(w32, 16), jnp.float32)
        xhi = lax.bitcast_convert_type(
            jnp.bitwise_and(w32, jnp.int32(-65536)), jnp.float32)
        xf = jnp.concatenate([xlo, xhi], axis=1)
        h = lax.dot_general(xf, wg, (((1,), (1,)), ((), ())),
                            preferred_element_type=jnp.float32)
        gate = h[:, :F]
        proj = h[:, F:]
        a = gate / (1.0 + jnp.exp(-gate)) * proj
        out = lax.dot_general(
            a, wd, (((1,), (1,)), ((), ())),
            preferred_element_type=jnp.float32)
        ys_ref[...] = _pack_bf16(out)


def _gmm(tvalid, tslot, tgrp, tfirst, gexp, gvalid, xs, Wg, Wd):
    grid_spec = pltpu.PrefetchScalarGridSpec(
        num_scalar_prefetch=6,
        grid=(NT,),
        in_specs=[
            pl.BlockSpec((BT, D // 2), lambda i, *refs: (refs[1][i, 0], 0)),
            pl.BlockSpec(memory_space=pl.ANY),
            pl.BlockSpec(memory_space=pl.ANY),
        ],
        out_specs=pl.BlockSpec((BT, D // 2),
                               lambda i, *refs: (refs[1][i, 0], 0)),
        scratch_shapes=[
            pltpu.VMEM((NBUF, 2 * F, D), jnp.float32),
            pltpu.VMEM((NBUF, D, F), jnp.float32),
            pltpu.SemaphoreType.DMA((NBUF,)),
            pltpu.SemaphoreType.DMA((NBUF,)),
        ],
    )
    return pl.pallas_call(
        _gmm_body,
        grid_spec=grid_spec,
        out_shape=jax.ShapeDtypeStruct((NSLOT, D // 2), jnp.int32),
        compiler_params=pltpu.CompilerParams(
            dimension_semantics=("arbitrary",)),
    )(tvalid, tslot, tgrp, tfirst, gexp, gvalid, xs, Wg, Wd)


def _combine(ys, pos, wpair):
    mesh = plsc.VectorSubcoreMesh(core_axis_name="c", subcore_axis_name="s")

    @functools.partial(
        pl.kernel, mesh=mesh,
        out_type=jax.ShapeDtypeStruct((T, D), jnp.float32),
        scratch_types=[
            pltpu.VMEM((TPW, D // 2), jnp.int32),
            pltpu.VMEM((TPW, D // 2), jnp.int32),
            pltpu.VMEM((TPW, D), jnp.float32),
            pltpu.VMEM((TPW, K), jnp.int32),
            pltpu.VMEM((TPW, K), jnp.float32),
            pltpu.VMEM((TPW,), jnp.int32),
            pltpu.VMEM((TPW,), jnp.int32),
            pltpu.SemaphoreType.DMA,
            pltpu.SemaphoreType.DMA,
        ],
        compiler_params=pltpu.CompilerParams(needs_layout_passes=False),
    )
    def comb(ys_hbm, pos_hbm, w_hbm, y_hbm, r0_v, r1_v, out_v, praw_v,
             wraw_v, idx0_v, idx1_v, s0, s1):
        wid = lax.axis_index("s") * NC + lax.axis_index("c")
        tbase = wid * TPW
        pltpu.sync_copy(pos_hbm.at[pl.ds(tbase, TPW)], praw_v)
        pltpu.sync_copy(w_hbm.at[pl.ds(tbase, TPW)], wraw_v)
        lane = lax.broadcasted_iota(jnp.int32, (16,), 0)
        c0 = jnp.zeros((16,), jnp.int32)
        c1 = jnp.ones((16,), jnp.int32)
        for c in range(TPW // 16):
            idx0_v[pl.ds(16 * c, 16)] = plsc.load_gather(
                praw_v, [16 * c + lane, c0])
            idx1_v[pl.ds(16 * c, 16)] = plsc.load_gather(
                praw_v, [16 * c + lane, c1])
        cp0 = pltpu.async_copy(ys_hbm.at[idx0_v], r0_v, s0)
        cp1 = pltpu.async_copy(ys_hbm.at[idx1_v], r1_v, s1)
        cp0.wait()
        cp1.wait()

        msk = jnp.full((16,), -65536, jnp.int32)

        def tok(t, _):
            w0 = plsc.load_gather(
                wraw_v, [jnp.full((16,), t, jnp.int32), c0])
            w1 = plsc.load_gather(
                wraw_v, [jnp.full((16,), t, jnp.int32), c1])
            for c in range(D // 32):
                sl = pl.ds(16 * c, 16)
                a = r0_v[t, sl]
                b = r1_v[t, sl]
                alo = plsc.bitcast(lax.shift_left(a, 16), jnp.float32)
                blo = plsc.bitcast(lax.shift_left(b, 16), jnp.float32)
                ahi = plsc.bitcast(jnp.bitwise_and(a, msk), jnp.float32)
                bhi = plsc.bitcast(jnp.bitwise_and(b, msk), jnp.float32)
                out_v[t, sl] = alo * w0 + blo * w1
                out_v[t, pl.ds(D // 2 + 16 * c, 16)] = ahi * w0 + bhi * w1
            return 0

        lax.fori_loop(0, TPW, tok, 0)
        pltpu.sync_copy(out_v, y_hbm.at[pl.ds(tbase, TPW)])

    return comb(ys, pos, wpair)


@jax.jit
def kernel(x, router_w, Wg, Wd):
    hs = x.reshape(T, D)
    (pos, wpair, texp, tvalid, tslot, tgrp, tfirst, gexp,
     gvalid, xpk) = _route(hs, router_w)
    xs = _dispatch(xpk, pos)
    ys = _gmm(tvalid, tslot, tgrp, tfirst, gexp, gvalid, xs, Wg, Wd)
    y = _combine(ys, pos, wpair)
    return y.reshape(B, S, D)


# speedup vs baseline: 1.0351x; 1.0351x over previous
import functools

import jax
import jax.numpy as jnp
from jax import lax
from jax.experimental import pallas as pl
from jax.experimental.pallas import tpu as pltpu
from jax.experimental.pallas import tpu_sc as plsc

B, S, D = 1, 2048, 768
E, K, F = 16, 2, 384
T = B * S
P = K * T

BT = 128
NT = P // BT + E
NSLOT = NT * BT

NC, NS = 2, 16
NW = NC * NS
TPW = T // NW
CH = 512


def _pack_bf16(v):
    bits = lax.bitcast_convert_type(v, jnp.int32)
    rnd = bits + 0x7FFF + jnp.bitwise_and(
        lax.shift_right_logical(bits, 16), 1)
    bf = lax.shift_right_logical(rnd, 16)
    lo = bf[:, :D // 2]
    hi = bf[:, D // 2:]
    return jnp.bitwise_or(lo, lax.shift_left(hi, 16))


def _route_body(x_ref, rw_ref, pos_ref, w_ref, texp_ref, tvalid_ref,
                tslot_ref, tgrp_ref, tfirst_ref, gexp_ref, gvalid_ref,
                xpk_ref):
    x = x_ref[...]
    xpk_ref[...] = _pack_bf16(x)
    ids = lax.broadcasted_iota(jnp.int32, (T, E), 1)
    logits = lax.dot_general(x, rw_ref[...], (((1,), (1,)), ((), ())),
                             preferred_element_type=jnp.float32)
    m = jnp.max(logits, axis=-1, keepdims=True)
    p = jnp.exp(logits - m)
    p = p / jnp.sum(p, axis=-1, keepdims=True)
    m1 = jnp.max(p, axis=-1, keepdims=True)
    i1 = jnp.min(jnp.where(p == m1, ids, E), axis=-1, keepdims=True)
    p2 = jnp.where(ids == i1, -1.0, p)
    m2 = jnp.max(p2, axis=-1, keepdims=True)
    i2 = jnp.min(jnp.where(p2 == m2, ids, E), axis=-1, keepdims=True)
    s = m1 + m2 + 1e-20
    w_ref[...] = jnp.concatenate([m1 / s, m2 / s], axis=1)

    cnt = (jnp.where(ids == i1, 1.0, 0.0)
           + jnp.where(ids == i2, 1.0, 0.0))
    r_io = lax.broadcasted_iota(jnp.int32, (CH, CH), 0)
    c_io = lax.broadcasted_iota(jnp.int32, (CH, CH), 1)
    tri = jnp.where(r_io >= c_io, 1.0, 0.0)
    carry = jnp.zeros((1, E), jnp.float32)
    cex_chunks = []
    for ci in range(T // CH):
        blk = cnt[ci * CH:(ci + 1) * CH]
        inc = lax.dot_general(tri, blk, (((1,), (0,)), ((), ())),
                              preferred_element_type=jnp.float32) + carry
        cex_chunks.append(inc - blk)
        carry = inc[CH - 1:CH, :]
    cex = jnp.concatenate(cex_chunks, axis=0)
    counts = carry

    tiles = (counts.astype(jnp.int32) + (BT - 1)) // BT
    tiles_f = tiles.astype(jnp.float32)
    er_io = lax.broadcasted_iota(jnp.int32, (E, E), 0)
    ec_io = lax.broadcasted_iota(jnp.int32, (E, E), 1)
    tril = jnp.where(er_io < ec_io, 1.0, 0.0)
    tilebase = lax.dot_general(tiles_f, tril, (((1,), (0,)), ((), ())),
                               preferred_element_type=jnp.float32)
    slotbase = tilebase * BT

    def _sel(mat, idx_t1):
        src = mat if mat.shape[0] == T else jnp.broadcast_to(mat, (T, E))
        return jnp.sum(jnp.where(ids == idx_t1, src, 0.0),
                       axis=-1, keepdims=True)

    pos0 = _sel(slotbase, i1) + _sel(cex, i1)
    pos1 = _sel(slotbase, i2) + _sel(cex, i2)
    pos_ref[...] = jnp.concatenate([pos0, pos1], axis=1).astype(jnp.int32)

    n_io = lax.broadcasted_iota(jnp.int32, (NT, E), 0).astype(jnp.float32)
    e_io = lax.broadcasted_iota(jnp.int32, (NT, E), 1).astype(jnp.float32)
    tb = jnp.broadcast_to(tilebase, (NT, E))
    tl = jnp.broadcast_to(tiles_f, (NT, E))
    inrange = jnp.logical_and(n_io >= tb, n_io < tb + tl)
    te = jnp.sum(jnp.where(inrange, e_io, 0.0), axis=-1, keepdims=True)
    total_tiles = jnp.sum(tiles_f)
    e1_io = lax.broadcasted_iota(jnp.int32, (1, E), 1).astype(jnp.float32)
    te_last = jnp.max(jnp.where(tiles_f > 0, e1_io, 0.0))
    valid = n_io[:, :1] < total_tiles
    texp_ref[...] = jnp.where(valid, te, te_last).astype(jnp.int32)
    tvalid_ref[...] = valid.astype(jnp.int32)
    tslot_ref[...] = jnp.where(valid, n_io[:, :1],
                               total_tiles - 1.0).astype(jnp.int32)

    used = jnp.where(counts > 0, 1.0, 0.0)
    gidx = lax.dot_general(used, tril, (((1,), (0,)), ((), ())),
                           preferred_element_type=jnp.float32)
    ngroups = jnp.sum(used)
    g_io = lax.broadcasted_iota(jnp.int32, (E, E), 0).astype(jnp.float32)
    e2_io = lax.broadcasted_iota(jnp.int32, (E, E), 1).astype(jnp.float32)
    cond = jnp.logical_and(jnp.broadcast_to(gidx, (E, E)) == g_io,
                           jnp.broadcast_to(used, (E, E)) > 0)
    gexp_ref[...] = jnp.sum(jnp.where(cond, e2_io, 0.0), axis=-1,
                            keepdims=True).astype(jnp.int32)
    gvalid_ref[...] = (g_io[:, :1] < ngroups).astype(jnp.int32)
    tgrp_ref[...] = jnp.sum(
        jnp.where(inrange, jnp.broadcast_to(gidx, (NT, E)), 0.0),
        axis=-1, keepdims=True).astype(jnp.int32)
    tfirst_ref[...] = jnp.sum(
        jnp.where(jnp.logical_and(inrange, n_io == tb), 1.0, 0.0),
        axis=-1, keepdims=True).astype(jnp.int32)


def _route(hs, router_w):
    return pl.pallas_call(
        _route_body,
        out_shape=[
            jax.ShapeDtypeStruct((T, K), jnp.int32),
            jax.ShapeDtypeStruct((T, K), jnp.float32),
            jax.ShapeDtypeStruct((NT, 1), jnp.int32),
            jax.ShapeDtypeStruct((NT, 1), jnp.int32),
            jax.ShapeDtypeStruct((NT, 1), jnp.int32),
            jax.ShapeDtypeStruct((NT, 1), jnp.int32),
            jax.ShapeDtypeStruct((NT, 1), jnp.int32),
            jax.ShapeDtypeStruct((E, 1), jnp.int32),
            jax.ShapeDtypeStruct((E, 1), jnp.int32),
            jax.ShapeDtypeStruct((T, D // 2), jnp.int32),
        ],
    )(hs, router_w)


def _dispatch(xpk, pos):
    mesh = plsc.VectorSubcoreMesh(core_axis_name="c", subcore_axis_name="s")

    @functools.partial(
        pl.kernel, mesh=mesh,
        out_type=jax.ShapeDtypeStruct((NSLOT, D // 2), jnp.int32),
        scratch_types=[
            pltpu.VMEM((TPW, D // 2), jnp.int32),
            pltpu.VMEM((TPW, K), jnp.int32),
            pltpu.VMEM((TPW,), jnp.int32),
            pltpu.VMEM((TPW,), jnp.int32),
            pltpu.SemaphoreType.DMA,
            pltpu.SemaphoreType.DMA,
        ],
        compiler_params=pltpu.CompilerParams(needs_layout_passes=False),
    )
    def disp(x_hbm, pos_hbm, xs_hbm, rows_v, praw_v, idx0_v, idx1_v, s0, s1):
        wid = lax.axis_index("s") * NC + lax.axis_index("c")
        tbase = wid * TPW
        pltpu.sync_copy(x_hbm.at[pl.ds(tbase, TPW)], rows_v)
        pltpu.sync_copy(pos_hbm.at[pl.ds(tbase, TPW)], praw_v)
        lane = lax.broadcasted_iota(jnp.int32, (16,), 0)
        c0 = jnp.zeros((16,), jnp.int32)
        c1 = jnp.ones((16,), jnp.int32)
        for c in range(TPW // 16):
            idx0_v[pl.ds(16 * c, 16)] = plsc.load_gather(
                praw_v, [16 * c + lane, c0])
            idx1_v[pl.ds(16 * c, 16)] = plsc.load_gather(
                praw_v, [16 * c + lane, c1])
        cp0 = pltpu.async_copy(rows_v, xs_hbm.at[idx0_v], s0)
        cp1 = pltpu.async_copy(rows_v, xs_hbm.at[idx1_v], s1)
        cp0.wait()
        cp1.wait()

    return disp(xpk, pos)


NBUF = 4


def _gmm_body(tvalid_ref, tslot_ref, tgrp_ref, tfirst_ref, gexp_ref,
              gvalid_ref, xs_ref, wg_hbm, wd_hbm, ys_ref, wgb, wdb, sg, sd):
    i = pl.program_id(0)

    def issue(g):
        gc = jnp.minimum(g, E - 1)
        e = gexp_ref[gc]
        b = lax.rem(gc, NBUF)

        @pl.when(jnp.logical_and(g < E, gvalid_ref[gc] == 1))
        def _():
            pltpu.make_async_copy(wg_hbm.at[e], wgb.at[b], sg.at[b]).start()
            pltpu.make_async_copy(wd_hbm.at[e], wdb.at[b], sd.at[b]).start()

    @pl.when(i == 0)
    def _():
        for g0 in range(NBUF - 1):
            issue(jnp.int32(g0))

    @pl.when(tvalid_ref[i] > 0)
    def _():
        g = tgrp_ref[i]
        b = lax.rem(g, NBUF)
        e = gexp_ref[g]

        @pl.when(tfirst_ref[i] == 1)
        def _():
            pltpu.make_async_copy(wg_hbm.at[e], wgb.at[b], sg.at[b]).wait()
            pltpu.make_async_copy(wd_hbm.at[e], wdb.at[b], sd.at[b]).wait()
            issue(g + NBUF - 1)

        wg = wgb[pl.ds(b, 1)][0]
        wd = wdb[pl.ds(b, 1)][0]
        w32 = xs_ref[...]
        xlo = lax.bitcast_convert_type(lax.shift_left(w32, 16), jnp.float32)
        xhi = lax.bitcast_convert_type(
            jnp.bitwise_and(w32, jnp.int32(-65536)), jnp.float32)
        xf = jnp.concatenate([xlo, xhi], axis=1)
        h = lax.dot_general(xf, wg, (((1,), (1,)), ((), ())),
                            preferred_element_type=jnp.float32)
        gate = h[:, :F]
        proj = h[:, F:]
        a = gate / (1.0 + jnp.exp(-gate)) * proj
        out = lax.dot_general(
            a, wd, (((1,), (1,)), ((), ())),
            preferred_element_type=jnp.float32)
        ys_ref[...] = _pack_bf16(out)


def _gmm(tvalid, tslot, tgrp, tfirst, gexp, gvalid, xs, Wg, Wd):
    grid_spec = pltpu.PrefetchScalarGridSpec(
        num_scalar_prefetch=6,
        grid=(NT,),
        in_specs=[
            pl.BlockSpec((BT, D // 2), lambda i, *refs: (refs[1][i], 0)),
            pl.BlockSpec(memory_space=pl.ANY),
            pl.BlockSpec(memory_space=pl.ANY),
        ],
        out_specs=pl.BlockSpec((BT, D // 2),
                               lambda i, *refs: (refs[1][i], 0)),
        scratch_shapes=[
            pltpu.VMEM((NBUF, 2 * F, D), jnp.float32),
            pltpu.VMEM((NBUF, D, F), jnp.float32),
            pltpu.SemaphoreType.DMA((NBUF,)),
            pltpu.SemaphoreType.DMA((NBUF,)),
        ],
    )
    return pl.pallas_call(
        _gmm_body,
        grid_spec=grid_spec,
        out_shape=jax.ShapeDtypeStruct((NSLOT, D // 2), jnp.int32),
        compiler_params=pltpu.CompilerParams(
            dimension_semantics=("arbitrary",)),
    )(tvalid.reshape(NT), tslot.reshape(NT), tgrp.reshape(NT),
      tfirst.reshape(NT), gexp.reshape(E), gvalid.reshape(E), xs, Wg, Wd)


def _combine(ys, pos, wpair):
    mesh = plsc.VectorSubcoreMesh(core_axis_name="c", subcore_axis_name="s")

    @functools.partial(
        pl.kernel, mesh=mesh,
        out_type=jax.ShapeDtypeStruct((T, D), jnp.float32),
        scratch_types=[
            pltpu.VMEM((TPW, D // 2), jnp.int32),
            pltpu.VMEM((TPW, D // 2), jnp.int32),
            pltpu.VMEM((TPW, D), jnp.float32),
            pltpu.VMEM((TPW, K), jnp.int32),
            pltpu.VMEM((TPW, K), jnp.float32),
            pltpu.VMEM((TPW,), jnp.int32),
            pltpu.VMEM((TPW,), jnp.int32),
            pltpu.SemaphoreType.DMA,
            pltpu.SemaphoreType.DMA,
        ],
        compiler_params=pltpu.CompilerParams(needs_layout_passes=False),
    )
    def comb(ys_hbm, pos_hbm, w_hbm, y_hbm, r0_v, r1_v, out_v, praw_v,
             wraw_v, idx0_v, idx1_v, s0, s1):
        wid = lax.axis_index("s") * NC + lax.axis_index("c")
        tbase = wid * TPW
        pltpu.sync_copy(pos_hbm.at[pl.ds(tbase, TPW)], praw_v)
        pltpu.sync_copy(w_hbm.at[pl.ds(tbase, TPW)], wraw_v)
        lane = lax.broadcasted_iota(jnp.int32, (16,), 0)
        c0 = jnp.zeros((16,), jnp.int32)
        c1 = jnp.ones((16,), jnp.int32)
        for c in range(TPW // 16):
            idx0_v[pl.ds(16 * c, 16)] = plsc.load_gather(
                praw_v, [16 * c + lane, c0])
            idx1_v[pl.ds(16 * c, 16)] = plsc.load_gather(
                praw_v, [16 * c + lane, c1])
        cp0 = pltpu.async_copy(ys_hbm.at[idx0_v], r0_v, s0)
        cp1 = pltpu.async_copy(ys_hbm.at[idx1_v], r1_v, s1)
        cp0.wait()
        cp1.wait()

        msk = jnp.full((16,), -65536, jnp.int32)

        def tok(t, _):
            w0 = plsc.load_gather(
                wraw_v, [jnp.full((16,), t, jnp.int32), c0])
            w1 = plsc.load_gather(
                wraw_v, [jnp.full((16,), t, jnp.int32), c1])
            for c in range(D // 32):
                sl = pl.ds(16 * c, 16)
                a = r0_v[t, sl]
                b = r1_v[t, sl]
                alo = plsc.bitcast(lax.shift_left(a, 16), jnp.float32)
                blo = plsc.bitcast(lax.shift_left(b, 16), jnp.float32)
                ahi = plsc.bitcast(jnp.bitwise_and(a, msk), jnp.float32)
                bhi = plsc.bitcast(jnp.bitwise_and(b, msk), jnp.float32)
                out_v[t, sl] = alo * w0 + blo * w1
                out_v[t, pl.ds(D // 2 + 16 * c, 16)] = ahi * w0 + bhi * w1
            return 0

        lax.fori_loop(0, TPW, tok, 0)
        pltpu.sync_copy(out_v, y_hbm.at[pl.ds(tbase, TPW)])

    return comb(ys, pos, wpair)


@jax.jit
def kernel(x, router_w, Wg, Wd):
    hs = x.reshape(T, D)
    (pos, wpair, texp, tvalid, tslot, tgrp, tfirst, gexp,
     gvalid, xpk) = _route(hs, router_w)
    xs = _dispatch(xpk, pos)
    ys = _gmm(tvalid, tslot, tgrp, tfirst, gexp, gvalid, xs, Wg, Wd)
    y = _combine(ys, pos, wpair)
    return y.reshape(B, S, D)
